# baseline (speedup 1.0000x reference)
import jax
import jax.numpy as jnp
from jax.experimental import pallas as pl
from jax.experimental.pallas import tpu as pltpu

BM1 = 400
BM2 = 800


def _layer1_body(x_ref, w1_ref, adj_ref, b_ref, wn_ref, o_ref, adjb_ref,
                 t_ref):
    @pl.when(pl.program_id(0) == 0)
    def _compute_t():
        t_ref[...] = jnp.dot(
            x_ref[...].astype(jnp.bfloat16), w1_ref[...],
            preferred_element_type=jnp.float32).astype(jnp.bfloat16)

    a = adj_ref[...].astype(jnp.bfloat16)
    adjb_ref[...] = a
    acc = jnp.dot(a, t_ref[...], preferred_element_type=jnp.float32)
    h = jnp.maximum(acc + b_ref[0, :], 0.0)
    o_ref[...] = jnp.dot(h.astype(jnp.bfloat16), wn_ref[...],
                         preferred_element_type=jnp.float32).astype(jnp.bfloat16)


def _l23_body(adj_ref, t2_ref, b2_ref, w3_ref, b3_ref, wd_ref, bd_ref,
              o_ref, t3_ref):
    p = pl.program_id(0)
    m = pl.program_id(1)
    bm = adj_ref.shape[0]

    @pl.when(p == 0)
    def _layer2():
        acc = jnp.dot(adj_ref[...], t2_ref[...],
                      preferred_element_type=jnp.float32)
        h = jnp.maximum(acc + b2_ref[0, :], 0.0)
        t3_ref[pl.ds(m * bm, bm), :] = jnp.dot(
            h.astype(jnp.bfloat16), w3_ref[...],
            preferred_element_type=jnp.float32).astype(jnp.bfloat16)

    @pl.when(p == 1)
    def _final():
        acc = jnp.dot(adj_ref[...], t3_ref[...],
                      preferred_element_type=jnp.float32)
        h = jnp.maximum(acc + b3_ref[0, :], 0.0)
        o_ref[...] = jnp.dot(h.astype(jnp.bfloat16), wd_ref[...],
                             preferred_element_type=jnp.float32) + bd_ref[0, :]


def _layer1(x, w1, adj, b, w_next):
    n, f = x.shape
    fo = w_next.shape[1]
    return pl.pallas_call(
        _layer1_body,
        grid=(pl.cdiv(n, BM1),),
        in_specs=[
            pl.BlockSpec((n, f), lambda m: (0, 0)),
            pl.BlockSpec((f, f), lambda m: (0, 0)),
            pl.BlockSpec((BM1, n), lambda m: (m, 0)),
            pl.BlockSpec((1, f), lambda m: (0, 0)),
            pl.BlockSpec((f, fo), lambda m: (0, 0)),
        ],
        out_specs=[
            pl.BlockSpec((BM1, fo), lambda m: (m, 0)),
            pl.BlockSpec((BM1, n), lambda m: (m, 0)),
        ],
        out_shape=[
            jax.ShapeDtypeStruct((n, fo), jnp.bfloat16),
            jax.ShapeDtypeStruct((n, n), jnp.bfloat16),
        ],
        scratch_shapes=[pltpu.VMEM((n, f), jnp.bfloat16)],
        compiler_params=pltpu.CompilerParams(
            dimension_semantics=("arbitrary",)),
    )(x, w1, adj, b, w_next)


def _l23(adj, t2, b2, w3, b3, wd, bd):
    n, f = t2.shape
    fo = wd.shape[1]
    return pl.pallas_call(
        _l23_body,
        grid=(2, pl.cdiv(n, BM2)),
        in_specs=[
            pl.BlockSpec((BM2, n), lambda p, m: (m, 0)),
            pl.BlockSpec((n, f), lambda p, m: (0, 0)),
            pl.BlockSpec((1, f), lambda p, m: (0, 0)),
            pl.BlockSpec((f, f), lambda p, m: (0, 0)),
            pl.BlockSpec((1, f), lambda p, m: (0, 0)),
            pl.BlockSpec((f, fo), lambda p, m: (0, 0)),
            pl.BlockSpec((1, fo), lambda p, m: (0, 0)),
        ],
        out_specs=pl.BlockSpec((BM2, fo), lambda p, m: (p * m, 0)),
        out_shape=jax.ShapeDtypeStruct((n, fo), jnp.float32),
        scratch_shapes=[pltpu.VMEM((n, f), jnp.bfloat16)],
        compiler_params=pltpu.CompilerParams(
            dimension_semantics=("arbitrary", "arbitrary")),
    )(adj, t2, b2, w3, b3, wd, bd)


def kernel(x, adj1, adj2, adj3, adj4, adj5, adj6, W1, b1, W2, b2, W3, b3,
           Wd, bd):
    w1b, w2b, w3b, wdb = (w.astype(jnp.bfloat16) for w in (W1, W2, W3, Wd))
    b1r = b1.reshape(1, -1)
    b2r = b2.reshape(1, -1)
    b3r = b3.reshape(1, -1)
    bdr = bd.reshape(1, -1)

    t2, adj_b = _layer1(x, w1b, adj5, b1r, w2b)
    return _l23(adj_b, t2, b2r, w3b, b3r, wdb, bdr)


# speedup vs baseline: 1.0155x; 1.0155x over previous
import jax
import jax.numpy as jnp
from jax.experimental import pallas as pl
from jax.experimental.pallas import tpu as pltpu

BM1 = 400
BM2 = 1000


def _layer1_body(x_ref, w1_ref, adj_ref, b_ref, wn_ref, o_ref, adjb_ref,
                 t_ref):
    @pl.when(pl.program_id(0) == 0)
    def _compute_t():
        t_ref[...] = jnp.dot(
            x_ref[...].astype(jnp.bfloat16), w1_ref[...],
            preferred_element_type=jnp.float32).astype(jnp.bfloat16)

    a = adj_ref[...].astype(jnp.bfloat16)
    adjb_ref[...] = a
    acc = jnp.dot(a, t_ref[...], preferred_element_type=jnp.float32)
    h = jnp.maximum(acc + b_ref[0, :], 0.0)
    o_ref[...] = jnp.dot(h.astype(jnp.bfloat16), wn_ref[...],
                         preferred_element_type=jnp.float32).astype(jnp.bfloat16)


def _l23_body(adj_ref, t2_ref, b2_ref, w3_ref, b3_ref, wd_ref, bd_ref,
              o_ref, t3_ref):
    p = pl.program_id(0)
    m = pl.program_id(1)
    bm = adj_ref.shape[0]

    @pl.when(p == 0)
    def _layer2():
        acc = jnp.dot(adj_ref[...], t2_ref[...],
                      preferred_element_type=jnp.float32)
        h = jnp.maximum(acc + b2_ref[0, :], 0.0)
        t3_ref[pl.ds(m * bm, bm), :] = jnp.dot(
            h.astype(jnp.bfloat16), w3_ref[...],
            preferred_element_type=jnp.float32).astype(jnp.bfloat16)

    @pl.when(p == 1)
    def _final():
        acc = jnp.dot(adj_ref[...], t3_ref[...],
                      preferred_element_type=jnp.float32)
        h = jnp.maximum(acc + b3_ref[0, :], 0.0)
        o_ref[...] = jnp.dot(h.astype(jnp.bfloat16), wd_ref[...],
                             preferred_element_type=jnp.float32) + bd_ref[0, :]


def _layer1(x, w1, adj, b, w_next):
    n, f = x.shape
    fo = w_next.shape[1]
    return pl.pallas_call(
        _layer1_body,
        grid=(pl.cdiv(n, BM1),),
        in_specs=[
            pl.BlockSpec((n, f), lambda m: (0, 0)),
            pl.BlockSpec((f, f), lambda m: (0, 0)),
            pl.BlockSpec((BM1, n), lambda m: (m, 0)),
            pl.BlockSpec((1, f), lambda m: (0, 0)),
            pl.BlockSpec((f, fo), lambda m: (0, 0)),
        ],
        out_specs=[
            pl.BlockSpec((BM1, fo), lambda m: (m, 0)),
            pl.BlockSpec((BM1, n), lambda m: (m, 0)),
        ],
        out_shape=[
            jax.ShapeDtypeStruct((n, fo), jnp.bfloat16),
            jax.ShapeDtypeStruct((n, n), jnp.bfloat16),
        ],
        scratch_shapes=[pltpu.VMEM((n, f), jnp.bfloat16)],
        compiler_params=pltpu.CompilerParams(
            dimension_semantics=("arbitrary",)),
    )(x, w1, adj, b, w_next)


def _l23(adj, t2, b2, w3, b3, wd, bd):
    n, f = t2.shape
    fo = wd.shape[1]
    return pl.pallas_call(
        _l23_body,
        grid=(2, pl.cdiv(n, BM2)),
        in_specs=[
            pl.BlockSpec((BM2, n), lambda p, m: (m, 0)),
            pl.BlockSpec((n, f), lambda p, m: (0, 0)),
            pl.BlockSpec((1, f), lambda p, m: (0, 0)),
            pl.BlockSpec((f, f), lambda p, m: (0, 0)),
            pl.BlockSpec((1, f), lambda p, m: (0, 0)),
            pl.BlockSpec((f, fo), lambda p, m: (0, 0)),
            pl.BlockSpec((1, fo), lambda p, m: (0, 0)),
        ],
        out_specs=pl.BlockSpec((BM2, fo), lambda p, m: (p * m, 0)),
        out_shape=jax.ShapeDtypeStruct((n, fo), jnp.float32),
        scratch_shapes=[pltpu.VMEM((n, f), jnp.bfloat16)],
        compiler_params=pltpu.CompilerParams(
            dimension_semantics=("arbitrary", "arbitrary")),
    )(adj, t2, b2, w3, b3, wd, bd)


def kernel(x, adj1, adj2, adj3, adj4, adj5, adj6, W1, b1, W2, b2, W3, b3,
           Wd, bd):
    w1b, w2b, w3b, wdb = (w.astype(jnp.bfloat16) for w in (W1, W2, W3, Wd))
    b1r = b1.reshape(1, -1)
    b2r = b2.reshape(1, -1)
    b3r = b3.reshape(1, -1)
    bdr = bd.reshape(1, -1)

    t2, adj_b = _layer1(x, w1b, adj5, b1r, w2b)
    return _l23(adj_b, t2, b2r, w3b, b3r, wdb, bdr)


# speedup vs baseline: 1.0340x; 1.0182x over previous
import jax
import jax.numpy as jnp
from jax.experimental import pallas as pl
from jax.experimental.pallas import tpu as pltpu

BM1 = 400
BM2 = 1000


def _layer1_body(x_ref, w1_ref, adj_ref, b_ref, wn_ref, o_ref, adjb_ref,
                 t_ref):
    @pl.when(pl.program_id(0) == 0)
    def _compute_t():
        t_ref[...] = jnp.dot(
            x_ref[...].astype(jnp.bfloat16), w1_ref[...],
            preferred_element_type=jnp.float32).astype(jnp.bfloat16)

    a = adj_ref[...].astype(jnp.bfloat16)
    adjb_ref[...] = a
    acc = jnp.dot(a, t_ref[...], preferred_element_type=jnp.float32)
    h = jnp.maximum(acc + b_ref[0, :], 0.0)
    o_ref[...] = jnp.dot(h.astype(jnp.bfloat16), wn_ref[...],
                         preferred_element_type=jnp.float32).astype(jnp.bfloat16)


def _l23_body(adj_ref, t2_ref, b2_ref, w3_ref, b3_ref, wd_ref, bd_ref,
              o_ref, t3_ref):
    p = pl.program_id(0)
    m = pl.program_id(1)
    bm = adj_ref.shape[0]

    @pl.when(p == 0)
    def _layer2():
        acc = jnp.dot(adj_ref[...], t2_ref[...],
                      preferred_element_type=jnp.float32)
        h = jnp.maximum(acc + b2_ref[0, :], 0.0)
        t3_ref[pl.ds(m * bm, bm), :] = jnp.dot(
            h.astype(jnp.bfloat16), w3_ref[...],
            preferred_element_type=jnp.float32).astype(jnp.bfloat16)

    @pl.when(p == 1)
    def _final():
        acc = jnp.dot(adj_ref[...], t3_ref[...],
                      preferred_element_type=jnp.float32)
        h = jnp.maximum(acc + b3_ref[0, :], 0.0)
        o_ref[...] = jnp.dot(h.astype(jnp.bfloat16), wd_ref[...],
                             preferred_element_type=jnp.float32) + bd_ref[0, :]


def _layer1(x, w1, adj, b, w_next):
    n, f = x.shape
    fo = w_next.shape[1]
    return pl.pallas_call(
        _layer1_body,
        grid=(pl.cdiv(n, BM1),),
        in_specs=[
            pl.BlockSpec((n, f), lambda m: (0, 0)),
            pl.BlockSpec((f, f), lambda m: (0, 0)),
            pl.BlockSpec((BM1, n), lambda m: (m, 0)),
            pl.BlockSpec((1, f), lambda m: (0, 0)),
            pl.BlockSpec((f, fo), lambda m: (0, 0)),
        ],
        out_specs=[
            pl.BlockSpec((BM1, fo), lambda m: (m, 0)),
            pl.BlockSpec((BM1, n), lambda m: (m, 0)),
        ],
        out_shape=[
            jax.ShapeDtypeStruct((n, fo), jnp.bfloat16),
            jax.ShapeDtypeStruct((n, n), jnp.bfloat16),
        ],
        scratch_shapes=[pltpu.VMEM((n, f), jnp.bfloat16)],
        compiler_params=pltpu.CompilerParams(
            dimension_semantics=("arbitrary",)),
    )(x, w1, adj, b, w_next)


def _l23(adj, t2, b2, w3, b3, wd, bd):
    n, f = t2.shape
    fo = wd.shape[1]
    nm = pl.cdiv(n, BM2)
    return pl.pallas_call(
        _l23_body,
        grid=(2, nm),
        in_specs=[
            pl.BlockSpec((BM2, n),
                         lambda p, m: (m + p * (nm - 1 - 2 * m), 0)),
            pl.BlockSpec((n, f), lambda p, m: (0, 0)),
            pl.BlockSpec((1, f), lambda p, m: (0, 0)),
            pl.BlockSpec((f, f), lambda p, m: (0, 0)),
            pl.BlockSpec((1, f), lambda p, m: (0, 0)),
            pl.BlockSpec((f, fo), lambda p, m: (0, 0)),
            pl.BlockSpec((1, fo), lambda p, m: (0, 0)),
        ],
        out_specs=pl.BlockSpec((BM2, fo), lambda p, m: (nm - 1 - p * m, 0)),
        out_shape=jax.ShapeDtypeStruct((n, fo), jnp.float32),
        scratch_shapes=[pltpu.VMEM((n, f), jnp.bfloat16)],
        compiler_params=pltpu.CompilerParams(
            dimension_semantics=("arbitrary", "arbitrary")),
    )(adj, t2, b2, w3, b3, wd, bd)


def kernel(x, adj1, adj2, adj3, adj4, adj5, adj6, W1, b1, W2, b2, W3, b3,
           Wd, bd):
    w1b, w2b, w3b, wdb = (w.astype(jnp.bfloat16) for w in (W1, W2, W3, Wd))
    b1r = b1.reshape(1, -1)
    b2r = b2.reshape(1, -1)
    b3r = b3.reshape(1, -1)
    bdr = bd.reshape(1, -1)

    t2, adj_b = _layer1(x, w1b, adj5, b1r, w2b)
    return _l23(adj_b, t2, b2r, w3b, b3r, wdb, bdr)
